# baseline (speedup 1.0000x reference)
import functools

import jax
import jax.numpy as jnp
from jax import lax
from jax.experimental import pallas as pl
from jax.experimental.pallas import tpu as pltpu
from jax.experimental.pallas import tpu_sc as plsc

N_FEATURES = 100000
N_HIDDEN = 16
BATCH = 1024

_NC, _NS = 2, 16
_NW = _NC * _NS
_BPW = BATCH // _NW


_WPW = _BPW * N_HIDDEN


def _sc_gather(buf, idx):
    mesh = plsc.VectorSubcoreMesh(core_axis_name="c", subcore_axis_name="s")

    ngrp = _WPW // 128

    @functools.partial(
        pl.kernel,
        mesh=mesh,
        out_type=jax.ShapeDtypeStruct((_NW * ngrp, 128), jnp.float32),
        scratch_types=[
            pltpu.VMEM((8 + _BPW,), jnp.int32),
            pltpu.VMEM((ngrp, 128), jnp.int32),
            pltpu.VMEM((ngrp, 128), jnp.float32),
            pltpu.SemaphoreType.DMA,
        ],
        compiler_params=pltpu.CompilerParams(
            use_tc_tiling_on_sc=False, needs_layout_passes=False,
            skip_device_barrier=True),
    )
    def gather_kernel(buf_hbm, idx_hbm, out_hbm, idx_v, widx_v, rows_v, sem):
        wid = lax.axis_index("s") * _NC + lax.axis_index("c")
        base = wid * _BPW
        pltpu.sync_copy(idx_hbm.at[pl.ds(base, _BPW)], idx_v.at[pl.ds(8, _BPW)])
        kofs = lax.iota(jnp.int32, 16) * N_FEATURES
        for r in range(_BPW):
            splat = plsc.load_gather(idx_v, [jnp.full((16,), 8 + r, jnp.int32)])
            g, o = divmod(r * N_HIDDEN, 128)
            widx_v[g, pl.ds(o, 16)] = splat + kofs
        copies = [
            pltpu.async_copy(buf_hbm.at[widx_v.at[g]], rows_v.at[g], sem)
            for g in range(ngrp)
        ]
        for c in copies:
            c.wait()
        pltpu.sync_copy(rows_v, out_hbm.at[pl.ds(wid * ngrp, ngrp)])

    return gather_kernel(buf, idx)


_VTILE = 2048


def _mm_body(wt_ref, xt_ref, o_ref):
    o_ref[...] = lax.dot_general(
        wt_ref[...], xt_ref[...],
        dimension_numbers=(((0,), (0,)), ((), ())),
        preferred_element_type=jnp.float32,
    )


def _tc_project_t(xt, Wt):
    grid = (pl.cdiv(N_FEATURES, _VTILE),)
    return pl.pallas_call(
        _mm_body,
        grid=grid,
        in_specs=[
            pl.BlockSpec((N_HIDDEN + 1, _VTILE), lambda i: (0, i)),
            pl.BlockSpec((N_HIDDEN + 1, BATCH), lambda i: (0, 0)),
        ],
        out_specs=pl.BlockSpec((_VTILE, BATCH), lambda i: (i, 0)),
        out_shape=jax.ShapeDtypeStruct((N_FEATURES, BATCH), jnp.float32),
    )(Wt, xt)


def kernel(data, emb, W, b):
    buf = emb.T.reshape(N_FEATURES * N_HIDDEN)
    x = _sc_gather(buf, data).reshape(BATCH, N_HIDDEN)
    xt = jnp.concatenate([x.T, jnp.ones((1, BATCH), jnp.float32)], axis=0)
    wt = jnp.concatenate([W.T, b[None, :]], axis=0)
    outT = _tc_project_t(xt, wt)
    return outT.T


# speedup vs baseline: 1.0107x; 1.0107x over previous
import functools

import jax
import jax.numpy as jnp
from jax import lax
from jax.experimental import pallas as pl
from jax.experimental.pallas import tpu as pltpu
from jax.experimental.pallas import tpu_sc as plsc

N_FEATURES = 100000
N_HIDDEN = 16
BATCH = 1024

_NC, _NS = 2, 16
_NW = _NC * _NS
_BPW = BATCH // _NW


_WPW = _BPW * N_HIDDEN


def _sc_gather(buf, idx):
    mesh = plsc.VectorSubcoreMesh(core_axis_name="c", subcore_axis_name="s")

    ngrp = _WPW // 128
    kpg = N_HIDDEN // ngrp

    @functools.partial(
        pl.kernel,
        mesh=mesh,
        out_type=jax.ShapeDtypeStruct((N_HIDDEN + 1, BATCH), jnp.float32),
        scratch_types=[
            pltpu.VMEM((_BPW,), jnp.int32),
            pltpu.VMEM((ngrp, 128), jnp.int32),
            pltpu.VMEM((ngrp, 128), jnp.float32),
            pltpu.VMEM((_BPW,), jnp.float32),
            pltpu.SemaphoreType.DMA,
        ],
        compiler_params=pltpu.CompilerParams(
            use_tc_tiling_on_sc=False, needs_layout_passes=False),
    )
    def gather_kernel(buf_hbm, idx_hbm, out_hbm, idx_v, widx_v, rows_v,
                      ones_v, sem):
        wid = lax.axis_index("s") * _NC + lax.axis_index("c")
        base = wid * _BPW
        pltpu.sync_copy(idx_hbm.at[pl.ds(base, _BPW)], idx_v)
        for h in range(_BPW // 16):
            ih = idx_v[pl.ds(h * 16, 16)]
            ones_v[pl.ds(h * 16, 16)] = jnp.full((16,), 1.0, jnp.float32)
            for g in range(ngrp):
                for kk in range(kpg):
                    widx_v[g, pl.ds(kk * _BPW + h * 16, 16)] = (
                        ih + (g * kpg + kk) * N_FEATURES)
        copies = [
            pltpu.async_copy(buf_hbm.at[widx_v.at[g]], rows_v.at[g], sem)
            for g in range(ngrp)
        ]
        for c in copies:
            c.wait()
        for g in range(ngrp):
            for kk in range(kpg):
                pltpu.sync_copy(
                    rows_v.at[g, pl.ds(kk * _BPW, _BPW)],
                    out_hbm.at[g * kpg + kk, pl.ds(base, _BPW)])
        pltpu.sync_copy(ones_v, out_hbm.at[N_HIDDEN, pl.ds(base, _BPW)])

    return gather_kernel(buf, idx)


_VTILE = 3072


def _mm_body(wt_ref, xt_ref, o_ref):
    o_ref[...] = lax.dot_general(
        wt_ref[...], xt_ref[...],
        dimension_numbers=(((0,), (0,)), ((), ())),
        preferred_element_type=jnp.float32,
    )


def _tc_project_t(xt, Wt):
    grid = (pl.cdiv(N_FEATURES, _VTILE),)
    return pl.pallas_call(
        _mm_body,
        grid=grid,
        in_specs=[
            pl.BlockSpec((N_HIDDEN + 1, _VTILE), lambda i: (0, i)),
            pl.BlockSpec((N_HIDDEN + 1, BATCH), lambda i: (0, 0)),
        ],
        out_specs=pl.BlockSpec((_VTILE, BATCH), lambda i: (i, 0)),
        out_shape=jax.ShapeDtypeStruct((N_FEATURES, BATCH), jnp.float32),
    )(Wt, xt)


def kernel(data, emb, W, b):
    buf = emb.T.reshape(N_FEATURES * N_HIDDEN)
    xt = _sc_gather(buf, data.astype(jnp.int32))
    wt = jnp.concatenate([W.T, b[None, :]], axis=0)
    outT = _tc_project_t(xt, wt)
    return outT.T
